# Initial kernel scaffold; baseline (speedup 1.0000x reference)
import jax
import jax.numpy as jnp
from jax.experimental import pallas as pl


def kernel(x, table_space, table_time):
    raise NotImplementedError("write your pallas kernel here")


# speedup vs baseline: 1.1294x; 1.1294x over previous
import functools

import numpy as np
import jax
import jax.numpy as jnp
from jax import lax
from jax.experimental import pallas as pl
from jax.experimental.pallas import tpu as pltpu
from jax.experimental.pallas import tpu_sc as plsc

NPTS = 524288
NLVL = 16
HSZ = 2 ** 19
MASK = HSZ - 1
P1 = np.int32(np.uint32(2654435761).view(np.int32))
P2 = np.int32(805459861)
RESOLUTIONS = [int(np.floor(16 * (1.4472692012786865 ** l))) for l in range(NLVL)]

NC = 2
NS = 16
NW = NC * NS
LANES = 16

CHUNK = 128
GROUPS = CHUNK // LANES
PT_PER_W = NPTS // NW
NCHUNK = PT_PER_W // CHUNK
ROWS = NLVL * 8 * CHUNK
NIDX = 2 * ROWS
NSTREAM = 4


def _add_body(a_ref, b_ref, o_ref):
    o_ref[...] = a_ref[...] + b_ref[...]


def _combine_tables(ts, tt):
    flat = NLVL * HSZ * 2
    a = ts.reshape(flat // 512, 512)
    b = tt.reshape(flat // 512, 512)
    blk = 1024
    out = pl.pallas_call(
        _add_body,
        grid=(flat // 512 // blk,),
        in_specs=[pl.BlockSpec((blk, 512), lambda i: (i, 0))] * 2,
        out_specs=pl.BlockSpec((blk, 512), lambda i: (i, 0)),
        out_shape=jax.ShapeDtypeStruct((flat // 512, 512), jnp.float32),
    )(a, b)
    return out.reshape(flat)


_MESH = plsc.VectorSubcoreMesh(core_axis_name="c", subcore_axis_name="s")


@functools.partial(
    pl.kernel,
    mesh=_MESH,
    compiler_params=pltpu.CompilerParams(
        needs_layout_passes=False, use_tc_tiling_on_sc=False),
    out_type=jax.ShapeDtypeStruct((NPTS * 32,), jnp.float32),
    scratch_types=[
        pltpu.VMEM((CHUNK * 3,), jnp.float32),
        pltpu.VMEM((NIDX,), jnp.int32),
        pltpu.VMEM((ROWS,), jnp.float32),
        pltpu.VMEM((NIDX,), jnp.float32),
        pltpu.VMEM((CHUNK * 32,), jnp.float32),
        pltpu.SemaphoreType.DMA,
    ],
)
def _encode(table_hbm, x_hbm, out_hbm, xv, idxv, wcv, rowsv, outv, sem):
    wid = lax.axis_index("s") * NC + lax.axis_index("c")
    lane = lax.iota(jnp.int32, 16)

    def chunk_body(ci, carry):
        base = wid * PT_PER_W + ci * CHUNK
        pltpu.sync_copy(x_hbm.at[pl.ds(base * 3, CHUNK * 3)], xv)

        def group_a(g, c):
            pids3 = (g * LANES + lane) * 3
            px = plsc.load_gather(xv, [pids3])
            py = plsc.load_gather(xv, [pids3 + 1])
            pz = plsc.load_gather(xv, [pids3 + 2])
            for l in range(NLVL):
                res = float(RESOLUTIONS[l])
                xs0 = px * res
                xs1 = py * res
                xs2 = pz * res
                i0 = xs0.astype(jnp.int32)
                i1 = xs1.astype(jnp.int32)
                i2 = xs2.astype(jnp.int32)
                w0 = xs0 - i0.astype(jnp.float32)
                w1 = xs1 - i1.astype(jnp.float32)
                w2 = xs2 - i2.astype(jnp.float32)
                v0 = 1.0 - w0
                v1 = 1.0 - w1
                v2 = 1.0 - w2
                m1 = i1 * P1
                m2 = i2 * P2
                m1b = m1 + P1
                m2b = m2 + P2
                i0b = i0 + 1
                for corner in range(8):
                    h = ((i0b if corner & 1 else i0)
                         ^ (m1b if corner & 2 else m1)
                         ^ (m2b if corner & 4 else m2))
                    hf = ((h & MASK) + l * HSZ) * 2
                    wc = ((w0 if corner & 1 else v0)
                          * (w1 if corner & 2 else v1)
                          * (w2 if corner & 4 else v2))
                    off2 = (l * 8 + corner) * 2 * CHUNK + g * 2 * LANES
                    off1 = (l * 8 + corner) * CHUNK + g * LANES
                    idxv[pl.ds(off2, LANES)] = hf
                    idxv[pl.ds(off2 + LANES, LANES)] = hf + 1
                    wcv[pl.ds(off1, LANES)] = wc
            return c

        lax.fori_loop(0, GROUPS, group_a, 0, unroll=False)

        qs = NIDX // NSTREAM
        cps = [
            pltpu.async_copy(
                table_hbm.at[idxv.at[pl.ds(q * qs, qs)]],
                rowsv.at[pl.ds(q * qs, qs)], sem)
            for q in range(NSTREAM)
        ]
        for cp in cps:
            cp.wait()

        def group_c(g, c):
            pids32 = (g * LANES + lane) * 32
            for l in range(NLVL):
                acc0 = jnp.zeros((16,), jnp.float32)
                acc1 = jnp.zeros((16,), jnp.float32)
                for corner in range(8):
                    off2 = (l * 8 + corner) * 2 * CHUNK + g * 2 * LANES
                    off1 = (l * 8 + corner) * CHUNK + g * LANES
                    wc = wcv[pl.ds(off1, LANES)]
                    f0 = rowsv[pl.ds(off2, LANES)]
                    f1 = rowsv[pl.ds(off2 + LANES, LANES)]
                    acc0 = acc0 + f0 * wc
                    acc1 = acc1 + f1 * wc
                plsc.store_scatter(outv, [pids32 + (2 * l)], acc0)
                plsc.store_scatter(outv, [pids32 + (2 * l + 1)], acc1)
            return c

        lax.fori_loop(0, GROUPS, group_c, 0, unroll=False)
        pltpu.sync_copy(outv, out_hbm.at[pl.ds(base * 32, CHUNK * 32)])
        return carry

    lax.fori_loop(0, NCHUNK, chunk_body, 0, unroll=False)


def kernel(x, table_space, table_time):
    table = _combine_tables(table_space, table_time)
    out = _encode(table, x.reshape(-1))
    return out.reshape(NPTS, 32)


# speedup vs baseline: 5.3841x; 4.7672x over previous
import functools

import numpy as np
import jax
import jax.numpy as jnp
from jax import lax
from jax.experimental import pallas as pl
from jax.experimental.pallas import tpu as pltpu
from jax.experimental.pallas import tpu_sc as plsc

NPTS = 524288
NLVL = 16
HSZ = 2 ** 19
MASK = HSZ - 1
P1 = np.int32(np.uint32(2654435761).view(np.int32))
P2 = np.int32(805459861)
RESOLUTIONS = [int(np.floor(16 * (1.4472692012786865 ** l))) for l in range(NLVL)]

NC = 2
NS = 16
NW = NC * NS
LANES = 16

CHUNK = 128
GROUPS = CHUNK // LANES
PT_PER_W = NPTS // NW
NCHUNK = PT_PER_W // CHUNK
ROWS = NLVL * 8 * CHUNK

HI_MASK = np.int32(-65536)


def _rne_bf16_hi(s):
    b = lax.bitcast_convert_type(s, jnp.int32)
    rb = b + np.int32(0x7FFF) + (lax.shift_right_logical(b, np.int32(16)) & np.int32(1))
    return rb & HI_MASK


def _pack_body(a0_ref, a1_ref, b0_ref, b1_ref, o_ref):
    u0 = _rne_bf16_hi(a0_ref[...] + b0_ref[...])
    u1 = _rne_bf16_hi(a1_ref[...] + b1_ref[...])
    o_ref[...] = lax.shift_right_logical(u0, np.int32(16)) | u1


def _combine_tables(ts, tt):
    n = NLVL * HSZ
    tsr = ts.reshape(n, 2)
    ttr = tt.reshape(n, 2)
    args = [a.reshape(n // 512, 512)
            for a in (tsr[:, 0], tsr[:, 1], ttr[:, 0], ttr[:, 1])]
    blk = 1024
    out = pl.pallas_call(
        _pack_body,
        grid=(n // 512 // blk,),
        in_specs=[pl.BlockSpec((blk, 512), lambda i: (i, 0))] * 4,
        out_specs=pl.BlockSpec((blk, 512), lambda i: (i, 0)),
        out_shape=jax.ShapeDtypeStruct((n // 512, 512), jnp.int32),
    )(*args)
    return out.reshape(n)


_MESH = plsc.VectorSubcoreMesh(core_axis_name="c", subcore_axis_name="s")


@functools.partial(
    pl.kernel,
    mesh=_MESH,
    compiler_params=pltpu.CompilerParams(
        needs_layout_passes=False, use_tc_tiling_on_sc=False),
    out_type=jax.ShapeDtypeStruct((NPTS * 32,), jnp.float32),
    scratch_types=[
        pltpu.VMEM((CHUNK * 3,), jnp.float32),
        pltpu.VMEM((ROWS,), jnp.int32),
        pltpu.VMEM((ROWS,), jnp.float32),
        pltpu.VMEM((ROWS,), jnp.int32),
        pltpu.VMEM((CHUNK * 32,), jnp.float32),
        pltpu.SemaphoreType.DMA,
    ],
)
def _encode(table_hbm, x_hbm, out_hbm, xv, idxv, wcv, rowsv, outv, sem):
    wid = lax.axis_index("s") * NC + lax.axis_index("c")
    lane = lax.iota(jnp.int32, 16)

    def chunk_body(ci, carry):
        base = wid * PT_PER_W + ci * CHUNK
        pltpu.sync_copy(x_hbm.at[pl.ds(base * 3, CHUNK * 3)], xv)

        def group_a(g, c):
            pids3 = (g * LANES + lane) * 3
            px = plsc.load_gather(xv, [pids3])
            py = plsc.load_gather(xv, [pids3 + 1])
            pz = plsc.load_gather(xv, [pids3 + 2])
            for l in range(NLVL):
                res = float(RESOLUTIONS[l])
                xs0 = px * res
                xs1 = py * res
                xs2 = pz * res
                i0 = xs0.astype(jnp.int32)
                i1 = xs1.astype(jnp.int32)
                i2 = xs2.astype(jnp.int32)
                w0 = xs0 - i0.astype(jnp.float32)
                w1 = xs1 - i1.astype(jnp.float32)
                w2 = xs2 - i2.astype(jnp.float32)
                v0 = 1.0 - w0
                v1 = 1.0 - w1
                v2 = 1.0 - w2
                m1 = i1 * P1
                m2 = i2 * P2
                m1b = m1 + P1
                m2b = m2 + P2
                i0b = i0 + 1
                for corner in range(8):
                    h = ((i0b if corner & 1 else i0)
                         ^ (m1b if corner & 2 else m1)
                         ^ (m2b if corner & 4 else m2))
                    hf = (h & MASK) + l * HSZ
                    wc = ((w0 if corner & 1 else v0)
                          * (w1 if corner & 2 else v1)
                          * (w2 if corner & 4 else v2))
                    off = (l * 8 + corner) * CHUNK + g * LANES
                    idxv[pl.ds(off, LANES)] = hf
                    wcv[pl.ds(off, LANES)] = wc
            return c

        lax.fori_loop(0, GROUPS, group_a, 0, unroll=False)

        pltpu.async_copy(table_hbm.at[idxv], rowsv, sem).wait()

        def group_c(g, c):
            pids32 = (g * LANES + lane) * 32
            for l in range(NLVL):
                acc0 = jnp.zeros((16,), jnp.float32)
                acc1 = jnp.zeros((16,), jnp.float32)
                for corner in range(8):
                    off = (l * 8 + corner) * CHUNK + g * LANES
                    wc = wcv[pl.ds(off, LANES)]
                    w = rowsv[pl.ds(off, LANES)]
                    f0 = plsc.bitcast(lax.shift_left(w, np.int32(16)), jnp.float32)
                    f1 = plsc.bitcast(w & HI_MASK, jnp.float32)
                    acc0 = acc0 + f0 * wc
                    acc1 = acc1 + f1 * wc
                plsc.store_scatter(outv, [pids32 + (2 * l)], acc0)
                plsc.store_scatter(outv, [pids32 + (2 * l + 1)], acc1)
            return c

        lax.fori_loop(0, GROUPS, group_c, 0, unroll=False)
        pltpu.sync_copy(outv, out_hbm.at[pl.ds(base * 32, CHUNK * 32)])
        return carry

    lax.fori_loop(0, NCHUNK, chunk_body, 0, unroll=False)


def kernel(x, table_space, table_time):
    table = _combine_tables(table_space, table_time)
    out = _encode(table, x.reshape(-1))
    return out.reshape(NPTS, 32)


# speedup vs baseline: 8.8041x; 1.6352x over previous
import functools

import numpy as np
import jax
import jax.numpy as jnp
from jax import lax
from jax.experimental import pallas as pl
from jax.experimental.pallas import tpu as pltpu
from jax.experimental.pallas import tpu_sc as plsc

NPTS = 524288
NLVL = 16
HSZ = 2 ** 19
MASK = HSZ - 1
P1 = np.int32(np.uint32(2654435761).view(np.int32))
P2 = np.int32(805459861)
RESOLUTIONS = [int(np.floor(16 * (1.4472692012786865 ** l))) for l in range(NLVL)]

NC = 2
NS = 16
NW = NC * NS
LANES = 16

CHUNK = 128
GROUPS = CHUNK // LANES
PT_PER_W = NPTS // NW
NCHUNK = PT_PER_W // CHUNK
ROWS = NLVL * 8 * CHUNK

HI_MASK = np.int32(-65536)


def _rne_bf16_hi(s):
    b = lax.bitcast_convert_type(s, jnp.int32)
    rb = b + np.int32(0x7FFF) + (lax.shift_right_logical(b, np.int32(16)) & np.int32(1))
    return rb & HI_MASK


def _pack_body(a_ref, b_ref, o_ref):
    s0 = a_ref[0, 0, :] + b_ref[0, 0, :]
    s1 = a_ref[0, 1, :] + b_ref[0, 1, :]
    o_ref[...] = lax.shift_right_logical(_rne_bf16_hi(s0), np.int32(16)) | _rne_bf16_hi(s1)


_TBLK = 8192


def _combine_tables(ts, tt):
    a = jnp.transpose(ts, (0, 2, 1))
    b = jnp.transpose(tt, (0, 2, 1))
    out = pl.pallas_call(
        _pack_body,
        grid=(NLVL, HSZ // _TBLK),
        in_specs=[pl.BlockSpec((1, 2, _TBLK), lambda i, j: (i, 0, j))] * 2,
        out_specs=pl.BlockSpec((_TBLK,), lambda i, j: (i * (HSZ // _TBLK) + j,)),
        out_shape=jax.ShapeDtypeStruct((NLVL * HSZ,), jnp.int32),
    )(a, b)
    return out


_PBLK = 8192


def _fold_body(*refs):
    o_ref = refs[-1]
    for c in range(32):
        o_ref[pl.ds(c, 1), :] = refs[c][...].reshape(1, _PBLK)


def _fold_out(flat):
    g = NPTS // _PBLK
    return pl.pallas_call(
        _fold_body,
        grid=(g,),
        in_specs=[pl.BlockSpec((_PBLK,), lambda i, c=c: (c * g + i,))
                  for c in range(32)],
        out_specs=pl.BlockSpec((32, _PBLK), lambda i: (0, i)),
        out_shape=jax.ShapeDtypeStruct((32, NPTS), jnp.float32),
    )(*([flat] * 32))


_MESH = plsc.VectorSubcoreMesh(core_axis_name="c", subcore_axis_name="s")


@functools.partial(
    pl.kernel,
    mesh=_MESH,
    compiler_params=pltpu.CompilerParams(
        needs_layout_passes=False, use_tc_tiling_on_sc=False),
    out_type=jax.ShapeDtypeStruct((NPTS * 32,), jnp.float32),
    scratch_types=[
        pltpu.VMEM((CHUNK * 3,), jnp.float32),
        pltpu.VMEM((ROWS,), jnp.int32),
        pltpu.VMEM((ROWS,), jnp.int32),
        pltpu.VMEM((ROWS,), jnp.float32),
        pltpu.VMEM((ROWS,), jnp.float32),
        pltpu.VMEM((ROWS,), jnp.int32),
        pltpu.VMEM((ROWS,), jnp.int32),
        pltpu.VMEM((CHUNK * 32,), jnp.float32),
        pltpu.SemaphoreType.DMA,
        pltpu.SemaphoreType.DMA,
        pltpu.SemaphoreType.DMA,
    ],
)
def _encode(table_hbm, x_hbm, out_hbm, xv,
            idx0, idx1, wc0, wc1, rows0, rows1, outv, sem0, sem1, sem_out):
    wid = lax.axis_index("s") * NC + lax.axis_index("c")
    lane = lax.iota(jnp.int32, 16)

    def a_phase(ci, idxv, wcv):
        base = wid * PT_PER_W + ci * CHUNK
        for d in range(3):
            pltpu.sync_copy(x_hbm.at[pl.ds(d * NPTS + base, CHUNK)],
                            xv.at[pl.ds(d * CHUNK, CHUNK)])

        def group_a(g, c):
            px = xv[pl.ds(0 * CHUNK + g * LANES, LANES)]
            py = xv[pl.ds(1 * CHUNK + g * LANES, LANES)]
            pz = xv[pl.ds(2 * CHUNK + g * LANES, LANES)]
            for l in range(NLVL):
                res = float(RESOLUTIONS[l])
                xs0 = px * res
                xs1 = py * res
                xs2 = pz * res
                i0 = xs0.astype(jnp.int32)
                i1 = xs1.astype(jnp.int32)
                i2 = xs2.astype(jnp.int32)
                w0 = xs0 - i0.astype(jnp.float32)
                w1 = xs1 - i1.astype(jnp.float32)
                w2 = xs2 - i2.astype(jnp.float32)
                v0 = 1.0 - w0
                v1 = 1.0 - w1
                v2 = 1.0 - w2
                m1 = i1 * P1
                m2 = i2 * P2
                m1b = m1 + P1
                m2b = m2 + P2
                i0b = i0 + 1
                for corner in range(8):
                    h = ((i0b if corner & 1 else i0)
                         ^ (m1b if corner & 2 else m1)
                         ^ (m2b if corner & 4 else m2))
                    hf = (h & MASK) + l * HSZ
                    wc = ((w0 if corner & 1 else v0)
                          * (w1 if corner & 2 else v1)
                          * (w2 if corner & 4 else v2))
                    off = (l * 8 + corner) * CHUNK + g * LANES
                    idxv[pl.ds(off, LANES)] = hf
                    wcv[pl.ds(off, LANES)] = wc
            return c

        lax.fori_loop(0, GROUPS, group_a, 0, unroll=False)

    def c_phase(ci, wcv, rowsv):
        base = wid * PT_PER_W + ci * CHUNK

        def group_c(g, c):
            for l in range(NLVL):
                acc0 = jnp.zeros((16,), jnp.float32)
                acc1 = jnp.zeros((16,), jnp.float32)
                for corner in range(8):
                    off = (l * 8 + corner) * CHUNK + g * LANES
                    wc = wcv[pl.ds(off, LANES)]
                    w = rowsv[pl.ds(off, LANES)]
                    f0 = plsc.bitcast(lax.shift_left(w, np.int32(16)), jnp.float32)
                    f1 = plsc.bitcast(w & HI_MASK, jnp.float32)
                    acc0 = acc0 + f0 * wc
                    acc1 = acc1 + f1 * wc
                outv[pl.ds((2 * l) * CHUNK + g * LANES, LANES)] = acc0
                outv[pl.ds((2 * l + 1) * CHUNK + g * LANES, LANES)] = acc1
            return c

        lax.fori_loop(0, GROUPS, group_c, 0, unroll=False)
        ocps = [
            pltpu.make_async_copy(
                outv.at[pl.ds(cc * CHUNK, CHUNK)],
                out_hbm.at[pl.ds(cc * NPTS + base, CHUNK)], sem_out)
            for cc in range(32)
        ]
        for cp in ocps:
            cp.start()
        for cp in ocps:
            cp.wait()

    def fire0():
        pltpu.make_async_copy(table_hbm.at[idx0], rows0, sem0).start()

    def fire1():
        pltpu.make_async_copy(table_hbm.at[idx1], rows1, sem1).start()

    a_phase(0, idx0, wc0)
    fire0()

    def pair_body(i, carry):
        e = 2 * i
        a_phase(e + 1, idx1, wc1)
        fire1()
        pltpu.make_async_copy(table_hbm.at[idx0], rows0, sem0).wait()
        c_phase(e, wc0, rows0)

        @pl.when(i < NCHUNK // 2 - 1)
        def _():
            a_phase(e + 2, idx0, wc0)
            fire0()

        pltpu.make_async_copy(table_hbm.at[idx1], rows1, sem1).wait()
        c_phase(e + 1, wc1, rows1)
        return carry

    lax.fori_loop(0, NCHUNK // 2, pair_body, 0, unroll=False)


def kernel(x, table_space, table_time):
    table = _combine_tables(table_space, table_time)
    xt = jnp.transpose(x).reshape(NPTS * 3)
    flat = _encode(table, xt)
    return jnp.transpose(_fold_out(flat))


# speedup vs baseline: 10.2326x; 1.1623x over previous
import functools

import numpy as np
import jax
import jax.numpy as jnp
from jax import lax
from jax.experimental import pallas as pl
from jax.experimental.pallas import tpu as pltpu
from jax.experimental.pallas import tpu_sc as plsc

NPTS = 524288
NLVL = 16
HSZ = 2 ** 19
MASK = HSZ - 1
P1 = np.int32(np.uint32(2654435761).view(np.int32))
P2 = np.int32(805459861)
RESOLUTIONS = [int(np.floor(16 * (1.4472692012786865 ** l))) for l in range(NLVL)]

NC = 2
NS = 16
NW = NC * NS
LANES = 16

CHUNK = 64
GROUPS = CHUNK // LANES
PT_PER_W = NPTS // NW
NCHUNK = PT_PER_W // CHUNK

NCL = 3
NSL = NLVL - NCL
ROWS = NSL * 8 * CHUNK

CR1 = [RESOLUTIONS[l] + 1 for l in range(NCL)]
CSZ = [r * r * r for r in CR1]
CPAD = [-(-s // 16) * 16 for s in CSZ]
COFF = [sum(CPAD[:i]) for i in range(NCL)]
CTOT = sum(CPAD)

HI_MASK = np.int32(-65536)


def _rne_bf16_hi(s):
    b = lax.bitcast_convert_type(s, jnp.int32)
    rb = b + np.int32(0x7FFF) + (lax.shift_right_logical(b, np.int32(16)) & np.int32(1))
    return rb & HI_MASK


def _pack_body(a_ref, b_ref, o_ref):
    s0 = a_ref[0, 0, :] + b_ref[0, 0, :]
    s1 = a_ref[0, 1, :] + b_ref[0, 1, :]
    o_ref[...] = lax.shift_right_logical(_rne_bf16_hi(s0), np.int32(16)) | _rne_bf16_hi(s1)


_TBLK = 8192


def _combine_tables(ts, tt):
    a = jnp.transpose(ts, (0, 2, 1))
    b = jnp.transpose(tt, (0, 2, 1))
    out = pl.pallas_call(
        _pack_body,
        grid=(NLVL, HSZ // _TBLK),
        in_specs=[pl.BlockSpec((1, 2, _TBLK), lambda i, j: (i, 0, j))] * 2,
        out_specs=pl.BlockSpec((_TBLK,), lambda i, j: (i * (HSZ // _TBLK) + j,)),
        out_shape=jax.ShapeDtypeStruct((NLVL * HSZ,), jnp.int32),
    )(a, b)
    return out


_PBLK = 8192


def _fold_body(*refs):
    o_ref = refs[-1]
    for c in range(32):
        o_ref[pl.ds(c, 1), :] = refs[c][...].reshape(1, _PBLK)


def _fold_out(flat):
    g = NPTS // _PBLK
    return pl.pallas_call(
        _fold_body,
        grid=(g,),
        in_specs=[pl.BlockSpec((_PBLK,), lambda i, c=c: (c * g + i,))
                  for c in range(32)],
        out_specs=pl.BlockSpec((32, _PBLK), lambda i: (0, i)),
        out_shape=jax.ShapeDtypeStruct((32, NPTS), jnp.float32),
    )(*([flat] * 32))


def _unpack(w):
    f0 = plsc.bitcast(lax.shift_left(w, np.int32(16)), jnp.float32)
    f1 = plsc.bitcast(w & HI_MASK, jnp.float32)
    return f0, f1


_MESH = plsc.VectorSubcoreMesh(core_axis_name="c", subcore_axis_name="s")


@functools.partial(
    pl.kernel,
    mesh=_MESH,
    compiler_params=pltpu.CompilerParams(
        needs_layout_passes=False, use_tc_tiling_on_sc=False),
    out_type=jax.ShapeDtypeStruct((NPTS * 32,), jnp.float32),
    scratch_types=[
        pltpu.VMEM((CHUNK * 3,), jnp.float32),
        pltpu.VMEM((CHUNK * 3,), jnp.float32),
        pltpu.VMEM((ROWS,), jnp.int32),
        pltpu.VMEM((ROWS,), jnp.int32),
        pltpu.VMEM((ROWS,), jnp.float32),
        pltpu.VMEM((ROWS,), jnp.float32),
        pltpu.VMEM((ROWS,), jnp.int32),
        pltpu.VMEM((ROWS,), jnp.int32),
        pltpu.VMEM((CHUNK * 32,), jnp.float32),
        pltpu.VMEM((CTOT,), jnp.int32),
        pltpu.SemaphoreType.DMA,
        pltpu.SemaphoreType.DMA,
        pltpu.SemaphoreType.DMA,
    ],
)
def _encode(table_hbm, x_hbm, out_hbm, xv0, xv1,
            idx0, idx1, wc0, wc1, rows0, rows1, outv, cachev,
            sem0, sem1, sem_out):
    wid = lax.axis_index("s") * NC + lax.axis_index("c")
    lane = lax.iota(jnp.int32, 16)

    for li in range(NCL):
        r1 = CR1[li]
        r1sq = r1 * r1
        inv1 = np.float32(1.0 / r1)
        inv2 = np.float32(1.0 / r1sq)
        last = np.int32(CSZ[li] - 1)
        for boff in range(0, CPAD[li], ROWS):
            n = min(ROWS, CPAD[li] - boff)

            def bgroup(g, c, boff=boff, r1=r1, r1sq=r1sq, inv1=inv1,
                       inv2=inv2, last=last, li=li):
                vid = jnp.minimum(boff + g * LANES + lane, last)
                vz = (vid.astype(jnp.float32) * inv2
                      + np.float32(1e-4)).astype(jnp.int32)
                remi = vid - vz * r1sq
                vy = (remi.astype(jnp.float32) * inv1
                      + np.float32(1e-4)).astype(jnp.int32)
                vx = remi - vy * r1
                h = ((vx ^ (vy * P1) ^ (vz * P2)) & MASK) + li * HSZ
                idx0[pl.ds(g * LANES, LANES)] = h
                return c

            lax.fori_loop(0, n // LANES, bgroup, 0, unroll=False)
            pltpu.async_copy(
                table_hbm.at[idx0.at[pl.ds(0, n)]],
                cachev.at[pl.ds(COFF[li] + boff, n)], sem0).wait()

    def a_phase(ci, idxv, wcv, xv):
        base = wid * PT_PER_W + ci * CHUNK
        for d in range(3):
            pltpu.sync_copy(x_hbm.at[pl.ds(d * NPTS + base, CHUNK)],
                            xv.at[pl.ds(d * CHUNK, CHUNK)])

        def group_a(g, c):
            px = xv[pl.ds(0 * CHUNK + g * LANES, LANES)]
            py = xv[pl.ds(1 * CHUNK + g * LANES, LANES)]
            pz = xv[pl.ds(2 * CHUNK + g * LANES, LANES)]
            for l in range(NCL, NLVL):
                res = float(RESOLUTIONS[l])
                xs0 = px * res
                xs1 = py * res
                xs2 = pz * res
                i0 = xs0.astype(jnp.int32)
                i1 = xs1.astype(jnp.int32)
                i2 = xs2.astype(jnp.int32)
                w0 = xs0 - i0.astype(jnp.float32)
                w1 = xs1 - i1.astype(jnp.float32)
                w2 = xs2 - i2.astype(jnp.float32)
                v0 = 1.0 - w0
                v1 = 1.0 - w1
                v2 = 1.0 - w2
                m1 = i1 * P1
                m2 = i2 * P2
                m1b = m1 + P1
                m2b = m2 + P2
                i0b = i0 + 1
                for corner in range(8):
                    h = ((i0b if corner & 1 else i0)
                         ^ (m1b if corner & 2 else m1)
                         ^ (m2b if corner & 4 else m2))
                    hf = (h & MASK) + l * HSZ
                    wc = ((w0 if corner & 1 else v0)
                          * (w1 if corner & 2 else v1)
                          * (w2 if corner & 4 else v2))
                    off = ((l - NCL) * 8 + corner) * CHUNK + g * LANES
                    idxv[pl.ds(off, LANES)] = hf
                    wcv[pl.ds(off, LANES)] = wc
            return c

        lax.fori_loop(0, GROUPS, group_a, 0, unroll=False)

    def b_phase(xv):
        def group_b(g, c):
            px = xv[pl.ds(0 * CHUNK + g * LANES, LANES)]
            py = xv[pl.ds(1 * CHUNK + g * LANES, LANES)]
            pz = xv[pl.ds(2 * CHUNK + g * LANES, LANES)]
            for li in range(NCL):
                res = float(RESOLUTIONS[li])
                r1 = CR1[li]
                r1sq = r1 * r1
                xs0 = px * res
                xs1 = py * res
                xs2 = pz * res
                i0 = xs0.astype(jnp.int32)
                i1 = xs1.astype(jnp.int32)
                i2 = xs2.astype(jnp.int32)
                w0 = xs0 - i0.astype(jnp.float32)
                w1 = xs1 - i1.astype(jnp.float32)
                w2 = xs2 - i2.astype(jnp.float32)
                v0 = 1.0 - w0
                v1 = 1.0 - w1
                v2 = 1.0 - w2
                d000 = i0 + i1 * r1 + i2 * r1sq + np.int32(COFF[li])
                acc0 = jnp.zeros((16,), jnp.float32)
                acc1 = jnp.zeros((16,), jnp.float32)
                for corner in range(8):
                    b0 = corner & 1
                    b1 = (corner >> 1) & 1
                    b2 = (corner >> 2) & 1
                    dc = d000 + np.int32(b0 + b1 * r1 + b2 * r1sq)
                    wc = ((w0 if b0 else v0)
                          * (w1 if b1 else v1)
                          * (w2 if b2 else v2))
                    f0, f1 = _unpack(plsc.load_gather(cachev, [dc]))
                    acc0 = acc0 + f0 * wc
                    acc1 = acc1 + f1 * wc
                outv[pl.ds((2 * li) * CHUNK + g * LANES, LANES)] = acc0
                outv[pl.ds((2 * li + 1) * CHUNK + g * LANES, LANES)] = acc1
            return c

        lax.fori_loop(0, GROUPS, group_b, 0, unroll=False)

    def c_phase(ci, wcv, rowsv):
        base = wid * PT_PER_W + ci * CHUNK

        def group_c(g, c):
            for l in range(NCL, NLVL):
                acc0 = jnp.zeros((16,), jnp.float32)
                acc1 = jnp.zeros((16,), jnp.float32)
                for corner in range(8):
                    off = ((l - NCL) * 8 + corner) * CHUNK + g * LANES
                    wc = wcv[pl.ds(off, LANES)]
                    f0, f1 = _unpack(rowsv[pl.ds(off, LANES)])
                    acc0 = acc0 + f0 * wc
                    acc1 = acc1 + f1 * wc
                outv[pl.ds((2 * l) * CHUNK + g * LANES, LANES)] = acc0
                outv[pl.ds((2 * l + 1) * CHUNK + g * LANES, LANES)] = acc1
            return c

        lax.fori_loop(0, GROUPS, group_c, 0, unroll=False)
        ocps = [
            pltpu.make_async_copy(
                outv.at[pl.ds(cc * CHUNK, CHUNK)],
                out_hbm.at[pl.ds(cc * NPTS + base, CHUNK)], sem_out)
            for cc in range(32)
        ]
        for cp in ocps:
            cp.start()
        for cp in ocps:
            cp.wait()

    def fire0():
        pltpu.make_async_copy(table_hbm.at[idx0], rows0, sem0).start()

    def fire1():
        pltpu.make_async_copy(table_hbm.at[idx1], rows1, sem1).start()

    a_phase(0, idx0, wc0, xv0)
    fire0()

    def pair_body(i, carry):
        e = 2 * i
        a_phase(e + 1, idx1, wc1, xv1)
        fire1()
        b_phase(xv0)
        pltpu.make_async_copy(table_hbm.at[idx0], rows0, sem0).wait()
        c_phase(e, wc0, rows0)

        @pl.when(i < NCHUNK // 2 - 1)
        def _():
            a_phase(e + 2, idx0, wc0, xv0)
            fire0()

        b_phase(xv1)
        pltpu.make_async_copy(table_hbm.at[idx1], rows1, sem1).wait()
        c_phase(e + 1, wc1, rows1)
        return carry

    lax.fori_loop(0, NCHUNK // 2, pair_body, 0, unroll=False)


def kernel(x, table_space, table_time):
    table = _combine_tables(table_space, table_time)
    xt = jnp.transpose(x).reshape(NPTS * 3)
    flat = _encode(table, xt)
    return jnp.transpose(_fold_out(flat))


# speedup vs baseline: 10.2620x; 1.0029x over previous
import functools

import numpy as np
import jax
import jax.numpy as jnp
from jax import lax
from jax.experimental import pallas as pl
from jax.experimental.pallas import tpu as pltpu
from jax.experimental.pallas import tpu_sc as plsc

NPTS = 524288
NLVL = 16
HSZ = 2 ** 19
MASK = HSZ - 1
P1 = np.int32(np.uint32(2654435761).view(np.int32))
P2 = np.int32(805459861)
RESOLUTIONS = [int(np.floor(16 * (1.4472692012786865 ** l))) for l in range(NLVL)]

NC = 2
NS = 16
NW = NC * NS
LANES = 16

CHUNK = 64
GROUPS = CHUNK // LANES
PT_PER_W = NPTS // NW
NCHUNK = PT_PER_W // CHUNK

NCL = 3
NSL = NLVL - NCL
ROWS = NSL * 8 * CHUNK

CR1 = [RESOLUTIONS[l] + 1 for l in range(NCL)]
CSZ = [r * r * r for r in CR1]
CPAD = [-(-s // 16) * 16 for s in CSZ]
COFF = [sum(CPAD[:i]) for i in range(NCL)]
CTOT = sum(CPAD)

HI_MASK = np.int32(-65536)


def _rne_bf16_hi(s):
    b = lax.bitcast_convert_type(s, jnp.int32)
    rb = b + np.int32(0x7FFF) + (lax.shift_right_logical(b, np.int32(16)) & np.int32(1))
    return rb & HI_MASK


def _pack_body(a_ref, b_ref, o_ref):
    s0 = a_ref[0, 0, :] + b_ref[0, 0, :]
    s1 = a_ref[0, 1, :] + b_ref[0, 1, :]
    o_ref[...] = lax.shift_right_logical(_rne_bf16_hi(s0), np.int32(16)) | _rne_bf16_hi(s1)


_TBLK = 8192


def _combine_tables(ts, tt):
    a = jnp.transpose(ts, (0, 2, 1))
    b = jnp.transpose(tt, (0, 2, 1))
    out = pl.pallas_call(
        _pack_body,
        grid=(NLVL, HSZ // _TBLK),
        in_specs=[pl.BlockSpec((1, 2, _TBLK), lambda i, j: (i, 0, j))] * 2,
        out_specs=pl.BlockSpec((_TBLK,), lambda i, j: (i * (HSZ // _TBLK) + j,)),
        out_shape=jax.ShapeDtypeStruct((NLVL * HSZ,), jnp.int32),
    )(a, b)
    return out


_PBLK = 8192


def _fold_body(*refs):
    o_ref = refs[-1]
    for c in range(32):
        o_ref[pl.ds(c, 1), :] = refs[c][...].reshape(1, _PBLK)


def _fold_out(flat):
    g = NPTS // _PBLK
    return pl.pallas_call(
        _fold_body,
        grid=(g,),
        in_specs=[pl.BlockSpec((_PBLK,), lambda i, c=c: (c * g + i,))
                  for c in range(32)],
        out_specs=pl.BlockSpec((32, _PBLK), lambda i: (0, i)),
        out_shape=jax.ShapeDtypeStruct((32, NPTS), jnp.float32),
    )(*([flat] * 32))


def _unpack(w):
    f0 = plsc.bitcast(lax.shift_left(w, np.int32(16)), jnp.float32)
    f1 = plsc.bitcast(w & HI_MASK, jnp.float32)
    return f0, f1


_MESH = plsc.VectorSubcoreMesh(core_axis_name="c", subcore_axis_name="s")


@functools.partial(
    pl.kernel,
    mesh=_MESH,
    compiler_params=pltpu.CompilerParams(
        needs_layout_passes=False, use_tc_tiling_on_sc=False),
    out_type=jax.ShapeDtypeStruct((NPTS * 32,), jnp.float32),
    scratch_types=[
        pltpu.VMEM((CHUNK * 3,), jnp.float32),
        pltpu.VMEM((CHUNK * 3,), jnp.float32),
        pltpu.VMEM((ROWS,), jnp.int32),
        pltpu.VMEM((ROWS,), jnp.int32),
        pltpu.VMEM((ROWS,), jnp.float32),
        pltpu.VMEM((ROWS,), jnp.float32),
        pltpu.VMEM((ROWS,), jnp.int32),
        pltpu.VMEM((ROWS,), jnp.int32),
        pltpu.VMEM((CHUNK * 32,), jnp.float32),
        pltpu.VMEM((CTOT,), jnp.int32),
        pltpu.SemaphoreType.DMA,
        pltpu.SemaphoreType.DMA,
        pltpu.SemaphoreType.DMA,
    ],
)
def _encode(table_hbm, x_hbm, out_hbm, xv0, xv1,
            idx0, idx1, wc0, wc1, rows0, rows1, outv, cachev,
            sem0, sem1, sem_out):
    wid = lax.axis_index("s") * NC + lax.axis_index("c")
    lane = lax.iota(jnp.int32, 16)

    batches = []
    for li in range(NCL):
        for boff in range(0, CPAD[li], ROWS):
            batches.append((li, boff, min(ROWS, CPAD[li] - boff)))

    def _bcopy(k):
        li, boff, n = batches[k]
        buf = idx0 if k % 2 == 0 else idx1
        sem = sem0 if k % 2 == 0 else sem1
        return pltpu.make_async_copy(
            table_hbm.at[buf.at[pl.ds(0, n)]],
            cachev.at[pl.ds(COFF[li] + boff, n)], sem)

    for k, (li, boff, n) in enumerate(batches):
        r1 = CR1[li]
        r1sq = r1 * r1
        inv1 = np.float32(1.0 / r1)
        inv2 = np.float32(1.0 / r1sq)
        last = np.int32(CSZ[li] - 1)
        buf = idx0 if k % 2 == 0 else idx1
        if k >= 2:
            _bcopy(k - 2).wait()

        def bgroup(g, c, boff=boff, r1=r1, r1sq=r1sq, inv1=inv1,
                   inv2=inv2, last=last, li=li, buf=buf):
            vid = jnp.minimum(boff + g * LANES + lane, last)
            vz = (vid.astype(jnp.float32) * inv2
                  + np.float32(1e-4)).astype(jnp.int32)
            remi = vid - vz * r1sq
            vy = (remi.astype(jnp.float32) * inv1
                  + np.float32(1e-4)).astype(jnp.int32)
            vx = remi - vy * r1
            h = ((vx ^ (vy * P1) ^ (vz * P2)) & MASK) + li * HSZ
            buf[pl.ds(g * LANES, LANES)] = h
            return c

        lax.fori_loop(0, n // LANES, bgroup, 0, unroll=False)
        _bcopy(k).start()
    for k in range(max(0, len(batches) - 2), len(batches)):
        _bcopy(k).wait()

    def a_phase(ci, idxv, wcv, xv):
        base = wid * PT_PER_W + ci * CHUNK
        for d in range(3):
            pltpu.sync_copy(x_hbm.at[pl.ds(d * NPTS + base, CHUNK)],
                            xv.at[pl.ds(d * CHUNK, CHUNK)])

        def group_a(g, c):
            px = xv[pl.ds(0 * CHUNK + g * LANES, LANES)]
            py = xv[pl.ds(1 * CHUNK + g * LANES, LANES)]
            pz = xv[pl.ds(2 * CHUNK + g * LANES, LANES)]
            for l in range(NCL, NLVL):
                res = float(RESOLUTIONS[l])
                xs0 = px * res
                xs1 = py * res
                xs2 = pz * res
                i0 = xs0.astype(jnp.int32)
                i1 = xs1.astype(jnp.int32)
                i2 = xs2.astype(jnp.int32)
                w0 = xs0 - i0.astype(jnp.float32)
                w1 = xs1 - i1.astype(jnp.float32)
                w2 = xs2 - i2.astype(jnp.float32)
                v0 = 1.0 - w0
                v1 = 1.0 - w1
                v2 = 1.0 - w2
                m1 = i1 * P1
                m2 = i2 * P2
                m1b = m1 + P1
                m2b = m2 + P2
                i0b = i0 + 1
                for corner in range(8):
                    h = ((i0b if corner & 1 else i0)
                         ^ (m1b if corner & 2 else m1)
                         ^ (m2b if corner & 4 else m2))
                    hf = (h & MASK) + l * HSZ
                    wc = ((w0 if corner & 1 else v0)
                          * (w1 if corner & 2 else v1)
                          * (w2 if corner & 4 else v2))
                    off = ((l - NCL) * 8 + corner) * CHUNK + g * LANES
                    idxv[pl.ds(off, LANES)] = hf
                    wcv[pl.ds(off, LANES)] = wc
            return c

        lax.fori_loop(0, GROUPS, group_a, 0, unroll=False)

    def b_phase(xv):
        def group_b(g, c):
            px = xv[pl.ds(0 * CHUNK + g * LANES, LANES)]
            py = xv[pl.ds(1 * CHUNK + g * LANES, LANES)]
            pz = xv[pl.ds(2 * CHUNK + g * LANES, LANES)]
            for li in range(NCL):
                res = float(RESOLUTIONS[li])
                r1 = CR1[li]
                r1sq = r1 * r1
                xs0 = px * res
                xs1 = py * res
                xs2 = pz * res
                i0 = xs0.astype(jnp.int32)
                i1 = xs1.astype(jnp.int32)
                i2 = xs2.astype(jnp.int32)
                w0 = xs0 - i0.astype(jnp.float32)
                w1 = xs1 - i1.astype(jnp.float32)
                w2 = xs2 - i2.astype(jnp.float32)
                v0 = 1.0 - w0
                v1 = 1.0 - w1
                v2 = 1.0 - w2
                d000 = i0 + i1 * r1 + i2 * r1sq + np.int32(COFF[li])
                acc0 = jnp.zeros((16,), jnp.float32)
                acc1 = jnp.zeros((16,), jnp.float32)
                for corner in range(8):
                    b0 = corner & 1
                    b1 = (corner >> 1) & 1
                    b2 = (corner >> 2) & 1
                    dc = d000 + np.int32(b0 + b1 * r1 + b2 * r1sq)
                    wc = ((w0 if b0 else v0)
                          * (w1 if b1 else v1)
                          * (w2 if b2 else v2))
                    f0, f1 = _unpack(plsc.load_gather(cachev, [dc]))
                    acc0 = acc0 + f0 * wc
                    acc1 = acc1 + f1 * wc
                outv[pl.ds((2 * li) * CHUNK + g * LANES, LANES)] = acc0
                outv[pl.ds((2 * li + 1) * CHUNK + g * LANES, LANES)] = acc1
            return c

        lax.fori_loop(0, GROUPS, group_b, 0, unroll=False)

    def c_phase(ci, wcv, rowsv):
        base = wid * PT_PER_W + ci * CHUNK

        def group_c(g, c):
            for l in range(NCL, NLVL):
                acc0 = jnp.zeros((16,), jnp.float32)
                acc1 = jnp.zeros((16,), jnp.float32)
                for corner in range(8):
                    off = ((l - NCL) * 8 + corner) * CHUNK + g * LANES
                    wc = wcv[pl.ds(off, LANES)]
                    f0, f1 = _unpack(rowsv[pl.ds(off, LANES)])
                    acc0 = acc0 + f0 * wc
                    acc1 = acc1 + f1 * wc
                outv[pl.ds((2 * l) * CHUNK + g * LANES, LANES)] = acc0
                outv[pl.ds((2 * l + 1) * CHUNK + g * LANES, LANES)] = acc1
            return c

        lax.fori_loop(0, GROUPS, group_c, 0, unroll=False)
        ocps = [
            pltpu.make_async_copy(
                outv.at[pl.ds(cc * CHUNK, CHUNK)],
                out_hbm.at[pl.ds(cc * NPTS + base, CHUNK)], sem_out)
            for cc in range(32)
        ]
        for cp in ocps:
            cp.start()
        for cp in ocps:
            cp.wait()

    def fire0():
        pltpu.make_async_copy(table_hbm.at[idx0], rows0, sem0).start()

    def fire1():
        pltpu.make_async_copy(table_hbm.at[idx1], rows1, sem1).start()

    a_phase(0, idx0, wc0, xv0)
    fire0()

    def pair_body(i, carry):
        e = 2 * i
        a_phase(e + 1, idx1, wc1, xv1)
        fire1()
        b_phase(xv0)
        pltpu.make_async_copy(table_hbm.at[idx0], rows0, sem0).wait()
        c_phase(e, wc0, rows0)

        @pl.when(i < NCHUNK // 2 - 1)
        def _():
            a_phase(e + 2, idx0, wc0, xv0)
            fire0()

        b_phase(xv1)
        pltpu.make_async_copy(table_hbm.at[idx1], rows1, sem1).wait()
        c_phase(e + 1, wc1, rows1)
        return carry

    lax.fori_loop(0, NCHUNK // 2, pair_body, 0, unroll=False)


def kernel(x, table_space, table_time):
    table = _combine_tables(table_space, table_time)
    xt = jnp.transpose(x).reshape(NPTS * 3)
    flat = _encode(table, xt)
    return jnp.transpose(_fold_out(flat))
